# baseline (speedup 1.0000x reference)
import functools

import jax
import jax.numpy as jnp
from jax import lax
from jax.experimental import pallas as pl
from jax.experimental.pallas import tpu as pltpu
from jax.experimental.pallas import tpu_sc as plsc

N_NODES = 10000
D_FEAT = 128
HIDDEN = 128
N_EDGES = 320000

_NC = 2
_NS = 16
_NW = _NC * _NS
_EPW = N_EDGES // _NW
_CHUNK = 80
_NCHUNK = _EPW // _CHUNK
_CPB = 25
_KBLK = _NCHUNK // _CPB
_NPAD = 10240
_STRIPE = _NPAD // _NS
_ZROWS = 64
_CNTW = 16

_ROWBLK = 400
_NBLK = N_NODES // _ROWBLK


def _fill_idx(idx_v, start):
    for i in range(_CHUNK // 16):
        idx_v[pl.ds(i * 16, 16)] = lax.iota(jnp.int32, 16) + start + i * 16


_DEPTH = 4


def _seg_body(with_cnt, y_hbm, src_hbm, dst_hbm, seg_out, *rest):
    if with_cnt:
        (cnt_out, acc_sh, cnt_sh, stg0, stg1, isem, didx, sidx, rows, gsem,
         ssem, ones_v, csem, i128_v, cbuf_v) = rest
    else:
        (acc_sh, stg0, stg1, isem, didx, sidx, rows, gsem, ssem) = rest
    cid = lax.axis_index("c")
    sid = lax.axis_index("s")
    wid = cid * _NS + sid
    base = sid * _STRIPE
    woff = wid * _EPW

    def _zb(i, c):
        rows[0][i // 8, pl.ds((i % 8) * 16, 16)] = jnp.zeros((16,),
                                                             jnp.float32)
        return c
    lax.fori_loop(0, _CHUNK * 8, _zb, 0)
    if with_cnt:
        for i in range(_CHUNK // 16):
            ones_v[pl.ds(i * 16, 16)] = jnp.zeros((16,), jnp.float32)
    for k in range(_STRIPE // _CHUNK):
        u = k % _DEPTH
        if k >= _DEPTH:
            pltpu.make_async_copy(rows[0], acc_sh.at[didx[u]],
                                  ssem[u]).wait()
            if with_cnt:
                pltpu.make_async_copy(ones_v, cnt_sh.at[didx[u]],
                                      csem[u]).wait()
        _fill_idx(didx[u], base + k * _CHUNK)
        pltpu.async_copy(rows[0], acc_sh.at[didx[u]], ssem[u])
        if with_cnt:
            pltpu.async_copy(ones_v, cnt_sh.at[didx[u]], csem[u])
    for u in range(_DEPTH):
        pltpu.make_async_copy(rows[0], acc_sh.at[didx[u]], ssem[u]).wait()
        if with_cnt:
            pltpu.make_async_copy(ones_v, cnt_sh.at[didx[u]],
                                  csem[u]).wait()
    if with_cnt:
        for i in range(_CHUNK // 16):
            ones_v[pl.ds(i * 16, 16)] = jnp.ones((16,), jnp.float32)

    plsc.subcore_barrier()

    blk = _DEPTH * _CHUNK
    stg = (stg0, stg1)

    def _stage(g, p):
        pltpu.async_copy(dst_hbm.at[pl.ds(woff + g * blk, blk)],
                         stg[p][0], isem[p])
        pltpu.async_copy(src_hbm.at[pl.ds(woff + g * blk, blk)],
                         stg[p][1], isem[p])

    def _stage_wait(g, p):
        pltpu.make_async_copy(dst_hbm.at[pl.ds(woff + g * blk, blk)],
                              stg[p][0], isem[p]).wait()
        pltpu.make_async_copy(src_hbm.at[pl.ds(woff + g * blk, blk)],
                              stg[p][1], isem[p]).wait()

    def _unpack(p, u):
        for i in range(_CHUNK // 16):
            didx[u][pl.ds(i * 16, 16)] = stg[p][0][pl.ds(u * _CHUNK
                                                         + i * 16, 16)]
            sidx[u][pl.ds(i * 16, 16)] = stg[p][1][pl.ds(u * _CHUNK
                                                         + i * 16, 16)]

    def _gather(u):
        pltpu.async_copy(y_hbm.at[sidx[u]], rows[u], gsem[u])

    def _gather_wait(u):
        pltpu.make_async_copy(y_hbm.at[sidx[u]], rows[u], gsem[u]).wait()

    def _scat(u):
        pltpu.async_copy(rows[u], acc_sh.at[didx[u]], ssem[u], add=True)
        if with_cnt:
            pltpu.async_copy(ones_v, cnt_sh.at[didx[u]], csem[u], add=True)

    def _scat_wait(u):
        pltpu.make_async_copy(rows[u], acc_sh.at[didx[u]], ssem[u]).wait()
        if with_cnt:
            pltpu.make_async_copy(ones_v, cnt_sh.at[didx[u]], csem[u]).wait()

    _stage(0, 0)
    _stage_wait(0, 0)
    for u in range(_DEPTH):
        _unpack(0, u)
        _gather(u)
    _stage(1, 1)
    for u in range(_DEPTH):
        _gather_wait(u)
        _scat(u)

    def _block(g, p, last=False):
        _stage_wait(g, p)
        for u in range(_DEPTH):
            _scat_wait(u)
            _unpack(p, u)
            _gather(u)
        if not last:
            _stage(g + 1, 1 - p)
        for u in range(_DEPTH):
            _gather_wait(u)
            _scat(u)

    nblk = _NCHUNK // _DEPTH

    def _body(h, c):
        g = 2 * h + 1
        _block(g, 1)
        _block(g + 1, 0)
        return c
    lax.fori_loop(0, (nblk - 3) // 2, _body, 0)
    _block(nblk - 2, 1)
    _block(nblk - 1, 0, last=True)

    for u in range(_NCHUNK % _DEPTH):
        j = nblk * _DEPTH + u
        _scat_wait(u)
        pltpu.sync_copy(dst_hbm.at[pl.ds(woff + j * _CHUNK, _CHUNK)],
                        didx[u])
        pltpu.sync_copy(src_hbm.at[pl.ds(woff + j * _CHUNK, _CHUNK)],
                        sidx[u])
        _gather(u)
    for u in range(_NCHUNK % _DEPTH, _DEPTH):
        _scat_wait(u)
    for u in range(_NCHUNK % _DEPTH):
        _gather_wait(u)
        _scat(u)
        _scat_wait(u)

    plsc.subcore_barrier()

    nw = _STRIPE // _CHUNK
    for k in range(nw):
        _fill_idx(didx[k % _DEPTH], base + k * _CHUNK)
        pltpu.async_copy(acc_sh.at[didx[k % _DEPTH]], rows[k % _DEPTH],
                         gsem[k % _DEPTH])
        if k % _DEPTH == _DEPTH - 1 or k == nw - 1:
            for kk in range(k - k % _DEPTH, k + 1):
                u = kk % _DEPTH
                pltpu.make_async_copy(acc_sh.at[didx[u]], rows[u],
                                      gsem[u]).wait()
                pltpu.sync_copy(rows[u],
                                seg_out.at[cid, pl.ds(base + kk * _CHUNK,
                                                      _CHUNK)])
    if with_cnt:
        nk = _STRIPE // 128
        for k in range(nk + 1):
            b = k % 2
            if k < nk:
                r = sid * nk + k
                for i in range(8):
                    i128_v[b][pl.ds(i * 16, 16)] = (lax.iota(jnp.int32, 16)
                                                    + r * 128 + i * 16)
                pltpu.async_copy(cnt_sh.at[i128_v[b]], cbuf_v[b], gsem[b])
            if k > 0:
                pb = 1 - b
                pltpu.make_async_copy(cnt_sh.at[i128_v[pb]], cbuf_v[pb],
                                      gsem[pb]).wait()
                pltpu.sync_copy(cbuf_v[pb],
                                cnt_out.at[cid, sid * nk + k - 1])


@functools.lru_cache(maxsize=None)
def _make_seg(with_cnt):
    mesh = plsc.VectorSubcoreMesh(core_axis_name="c", subcore_axis_name="s")
    out_type = [jax.ShapeDtypeStruct((_NC, _NPAD, D_FEAT), jnp.float32)]
    scratch = [pltpu.VMEM_SHARED((_NPAD, D_FEAT), jnp.float32)]
    if with_cnt:
        out_type.append(jax.ShapeDtypeStruct((_NC, _NPAD // 128, 128),
                                             jnp.float32))
        scratch.append(pltpu.VMEM_SHARED((_NPAD,), jnp.float32))
    scratch += [
        [pltpu.VMEM((_DEPTH * _CHUNK,), jnp.int32)] * 2,
        [pltpu.VMEM((_DEPTH * _CHUNK,), jnp.int32)] * 2,
        [pltpu.SemaphoreType.DMA] * 2,
        [pltpu.VMEM((_CHUNK,), jnp.int32)] * _DEPTH,
        [pltpu.VMEM((_CHUNK,), jnp.int32)] * _DEPTH,
        [pltpu.VMEM((_CHUNK, D_FEAT), jnp.float32)] * _DEPTH,
        [pltpu.SemaphoreType.DMA] * _DEPTH,
        [pltpu.SemaphoreType.DMA] * _DEPTH,
    ]
    if with_cnt:
        scratch += [
            pltpu.VMEM((_CHUNK,), jnp.float32),
            [pltpu.SemaphoreType.DMA] * _DEPTH,
            [pltpu.VMEM((128,), jnp.int32)] * 2,
            [pltpu.VMEM((128,), jnp.float32)] * 2,
        ]
    return pl.kernel(functools.partial(_seg_body, with_cnt),
                     out_type=out_type, mesh=mesh, scratch_types=scratch)


def _lin2_body(x_ref, wl_ref, wr_ref, b_ref, y_ref, r_ref):
    xb = x_ref[...]
    y_ref[...] = jnp.dot(xb, wl_ref[...], preferred_element_type=jnp.float32)
    r_ref[...] = (jnp.dot(xb, wr_ref[...], preferred_element_type=jnp.float32)
                  + b_ref[...])


def _mid_body(seg_ref, cnt_ref, r1_ref, wl_ref, wr_ref, b_ref, y_ref, r_ref):
    seg = seg_ref[0] + seg_ref[1]
    c = cnt_ref[0] + cnt_ref[1]
    inv = 1.0 / jnp.maximum(c, 1.0)
    h = jnp.maximum(seg * inv + r1_ref[...], 0.0)
    y_ref[...] = jnp.dot(h, wl_ref[...], preferred_element_type=jnp.float32)
    r_ref[...] = (jnp.dot(h, wr_ref[...], preferred_element_type=jnp.float32)
                  + b_ref[...])


def _fin_body(seg_ref, cnt_ref, r2_ref, out_ref):
    seg = seg_ref[0] + seg_ref[1]
    c = cnt_ref[0] + cnt_ref[1]
    inv = 1.0 / jnp.maximum(c, 1.0)
    out_ref[...] = seg * inv + r2_ref[...]


_rowspec = pl.BlockSpec((_ROWBLK, D_FEAT), lambda i: (i, 0))
_wspec = pl.BlockSpec((D_FEAT, HIDDEN), lambda i: (0, 0))
_bspec = pl.BlockSpec((1, HIDDEN), lambda i: (0, 0))
_segspec = pl.BlockSpec((_NC, _ROWBLK, D_FEAT), lambda i: (0, i, 0))
_cntspec = pl.BlockSpec((_NC, _ROWBLK, 1), lambda i: (0, i, 0))
_out2 = [jax.ShapeDtypeStruct((N_NODES, HIDDEN), jnp.float32)] * 2

_lin2 = pl.pallas_call(
    _lin2_body, grid=(_NBLK,),
    in_specs=[_rowspec, _wspec, _wspec, _bspec],
    out_specs=[_rowspec, _rowspec],
    out_shape=_out2,
)

_mid = pl.pallas_call(
    _mid_body, grid=(_NBLK,),
    in_specs=[_segspec, _cntspec, _rowspec, _wspec, _wspec, _bspec],
    out_specs=[_rowspec, _rowspec],
    out_shape=_out2,
)

_fin = pl.pallas_call(
    _fin_body, grid=(_NBLK,),
    in_specs=[_segspec, _cntspec, _rowspec],
    out_specs=_rowspec,
    out_shape=jax.ShapeDtypeStruct((N_NODES, HIDDEN), jnp.float32),
)


def kernel(x, edge_index, W1_l, W1_r, b1, W2_l, W2_r, b2):
    src = edge_index[0].astype(jnp.int32)
    dst = edge_index[1].astype(jnp.int32)
    b1r = b1.reshape(1, HIDDEN)
    b2r = b2.reshape(1, HIDDEN)

    y1, r1 = _lin2(x, W1_l, W1_r, b1r)
    seg1, cntp = _make_seg(True)(y1, src, dst)
    cnt = cntp.reshape(_NC, _NPAD, 1)
    y2, r2 = _mid(seg1, cnt, r1, W2_l, W2_r, b2r)
    (seg2,) = _make_seg(False)(y2, src, dst)
    out = _fin(seg2, cnt, r2)
    return out


# speedup vs baseline: 1.0000x; 1.0000x over previous
import functools

import jax
import jax.numpy as jnp
from jax import lax
from jax.experimental import pallas as pl
from jax.experimental.pallas import tpu as pltpu
from jax.experimental.pallas import tpu_sc as plsc

N_NODES = 10000
D_FEAT = 128
HIDDEN = 128
N_EDGES = 320000

_NC = 2
_NS = 16
_NW = _NC * _NS
_EPW = N_EDGES // _NW
_CHUNK = 80
_NCHUNK = _EPW // _CHUNK
_CPB = 25
_KBLK = _NCHUNK // _CPB
_NPAD = 10240
_STRIPE = _NPAD // _NS
_ZROWS = 64
_CNTW = 16

_ROWBLK = 400
_NBLK = N_NODES // _ROWBLK


def _fill_idx(idx_v, start):
    for i in range(_CHUNK // 16):
        idx_v[pl.ds(i * 16, 16)] = lax.iota(jnp.int32, 16) + start + i * 16


_DEPTH = 4


def _seg_body(with_cnt, y_hbm, src_hbm, dst_hbm, seg_out, *rest):
    if with_cnt:
        (cnt_out, acc_sh, cnt_sh, stg0, stg1, isem, didx, sidx, rows, gsem,
         ssem, ones_v, csem, i128_v, cbuf_v) = rest
    else:
        (acc_sh, stg0, stg1, isem, didx, sidx, rows, gsem, ssem) = rest
    cid = lax.axis_index("c")
    sid = lax.axis_index("s")
    wid = cid * _NS + sid
    base = sid * _STRIPE
    woff = wid * _EPW

    def _zb(i, c):
        rows[0][i // 8, pl.ds((i % 8) * 16, 16)] = jnp.zeros((16,),
                                                             jnp.float32)
        return c
    lax.fori_loop(0, _CHUNK * 8, _zb, 0)
    if with_cnt:
        for i in range(_CHUNK // 16):
            ones_v[pl.ds(i * 16, 16)] = jnp.zeros((16,), jnp.float32)
    for k in range(_STRIPE // _CHUNK):
        u = k % _DEPTH
        if k >= _DEPTH:
            pltpu.make_async_copy(rows[0], acc_sh.at[didx[u]],
                                  ssem[u]).wait()
            if with_cnt:
                pltpu.make_async_copy(ones_v, cnt_sh.at[didx[u]],
                                      csem[u]).wait()
        _fill_idx(didx[u], base + k * _CHUNK)
        pltpu.async_copy(rows[0], acc_sh.at[didx[u]], ssem[u])
        if with_cnt:
            pltpu.async_copy(ones_v, cnt_sh.at[didx[u]], csem[u])
    for u in range(_DEPTH):
        pltpu.make_async_copy(rows[0], acc_sh.at[didx[u]], ssem[u]).wait()
        if with_cnt:
            pltpu.make_async_copy(ones_v, cnt_sh.at[didx[u]],
                                  csem[u]).wait()
    if with_cnt:
        for i in range(_CHUNK // 16):
            ones_v[pl.ds(i * 16, 16)] = jnp.ones((16,), jnp.float32)

    plsc.subcore_barrier()

    blk = _DEPTH * _CHUNK
    stg = (stg0, stg1)

    def _stage(g, p):
        pltpu.async_copy(dst_hbm.at[pl.ds(woff + g * blk, blk)],
                         stg[p][0], isem[p])
        pltpu.async_copy(src_hbm.at[pl.ds(woff + g * blk, blk)],
                         stg[p][1], isem[p])

    def _stage_wait(g, p):
        pltpu.make_async_copy(dst_hbm.at[pl.ds(woff + g * blk, blk)],
                              stg[p][0], isem[p]).wait()
        pltpu.make_async_copy(src_hbm.at[pl.ds(woff + g * blk, blk)],
                              stg[p][1], isem[p]).wait()

    def _unpack(p, u):
        for i in range(_CHUNK // 16):
            didx[u][pl.ds(i * 16, 16)] = stg[p][0][pl.ds(u * _CHUNK
                                                         + i * 16, 16)]
            sidx[u][pl.ds(i * 16, 16)] = stg[p][1][pl.ds(u * _CHUNK
                                                         + i * 16, 16)]

    def _gather(u):
        pltpu.async_copy(y_hbm.at[sidx[u]], rows[u], gsem[u])

    def _gather_wait(u):
        pltpu.make_async_copy(y_hbm.at[sidx[u]], rows[u], gsem[u]).wait()

    def _scat(u):
        pltpu.async_copy(rows[u], acc_sh.at[didx[u]], ssem[u], add=True)
        if with_cnt:
            pltpu.async_copy(ones_v, cnt_sh.at[didx[u]], csem[u], add=True)

    def _scat_wait(u):
        pltpu.make_async_copy(rows[u], acc_sh.at[didx[u]], ssem[u]).wait()
        if with_cnt:
            pltpu.make_async_copy(ones_v, cnt_sh.at[didx[u]], csem[u]).wait()

    _stage(0, 0)
    _stage_wait(0, 0)
    for u in range(_DEPTH):
        _unpack(0, u)
        _gather(u)
    _stage(1, 1)
    for u in range(_DEPTH):
        _gather_wait(u)
        _scat(u)

    def _block(g, p, last=False):
        _stage_wait(g, p)
        for u in range(_DEPTH):
            _scat_wait(u)
            _unpack(p, u)
            _gather(u)
        if not last:
            _stage(g + 1, 1 - p)
        for u in range(_DEPTH):
            _gather_wait(u)
            _scat(u)

    nblk = _NCHUNK // _DEPTH

    def _body(h, c):
        g = 2 * h + 1
        _block(g, 1)
        _block(g + 1, 0)
        return c
    lax.fori_loop(0, (nblk - 3) // 2, _body, 0)
    _block(nblk - 2, 1)
    _block(nblk - 1, 0, last=True)

    for u in range(_NCHUNK % _DEPTH):
        j = nblk * _DEPTH + u
        _scat_wait(u)
        pltpu.sync_copy(dst_hbm.at[pl.ds(woff + j * _CHUNK, _CHUNK)],
                        didx[u])
        pltpu.sync_copy(src_hbm.at[pl.ds(woff + j * _CHUNK, _CHUNK)],
                        sidx[u])
        _gather(u)
    for u in range(_NCHUNK % _DEPTH, _DEPTH):
        _scat_wait(u)
    for u in range(_NCHUNK % _DEPTH):
        _gather_wait(u)
        _scat(u)
        _scat_wait(u)

    plsc.subcore_barrier()

    nw = _STRIPE // _CHUNK
    for k in range(nw):
        _fill_idx(didx[k % _DEPTH], base + k * _CHUNK)
        pltpu.async_copy(acc_sh.at[didx[k % _DEPTH]], rows[k % _DEPTH],
                         gsem[k % _DEPTH])
        if k % _DEPTH == _DEPTH - 1 or k == nw - 1:
            for kk in range(k - k % _DEPTH, k + 1):
                u = kk % _DEPTH
                pltpu.make_async_copy(acc_sh.at[didx[u]], rows[u],
                                      gsem[u]).wait()
                pltpu.sync_copy(rows[u],
                                seg_out.at[cid, pl.ds(base + kk * _CHUNK,
                                                      _CHUNK)])
    if with_cnt:
        nk = _STRIPE // 128
        for k in range(nk + 1):
            b = k % 2
            if k < nk:
                r = sid * nk + k
                for i in range(8):
                    i128_v[b][pl.ds(i * 16, 16)] = (lax.iota(jnp.int32, 16)
                                                    + r * 128 + i * 16)
                pltpu.async_copy(cnt_sh.at[i128_v[b]], cbuf_v[b], gsem[b])
            if k > 0:
                pb = 1 - b
                pltpu.make_async_copy(cnt_sh.at[i128_v[pb]], cbuf_v[pb],
                                      gsem[pb]).wait()
                pltpu.sync_copy(cbuf_v[pb],
                                cnt_out.at[cid, sid * nk + k - 1])


@functools.lru_cache(maxsize=None)
def _make_seg(with_cnt):
    mesh = plsc.VectorSubcoreMesh(core_axis_name="c", subcore_axis_name="s")
    out_type = [jax.ShapeDtypeStruct((_NC, _NPAD, D_FEAT), jnp.float32)]
    scratch = [pltpu.VMEM_SHARED((_NPAD, D_FEAT), jnp.float32)]
    if with_cnt:
        out_type.append(jax.ShapeDtypeStruct((_NC, _NPAD // 128, 128),
                                             jnp.float32))
        scratch.append(pltpu.VMEM_SHARED((_NPAD,), jnp.float32))
    scratch += [
        [pltpu.VMEM((_DEPTH * _CHUNK,), jnp.int32)] * 2,
        [pltpu.VMEM((_DEPTH * _CHUNK,), jnp.int32)] * 2,
        [pltpu.SemaphoreType.DMA] * 2,
        [pltpu.VMEM((_CHUNK,), jnp.int32)] * _DEPTH,
        [pltpu.VMEM((_CHUNK,), jnp.int32)] * _DEPTH,
        [pltpu.VMEM((_CHUNK, D_FEAT), jnp.float32)] * _DEPTH,
        [pltpu.SemaphoreType.DMA] * _DEPTH,
        [pltpu.SemaphoreType.DMA] * _DEPTH,
    ]
    if with_cnt:
        scratch += [
            pltpu.VMEM((_CHUNK,), jnp.float32),
            [pltpu.SemaphoreType.DMA] * _DEPTH,
            [pltpu.VMEM((128,), jnp.int32)] * 2,
            [pltpu.VMEM((128,), jnp.float32)] * 2,
        ]
    return pl.kernel(functools.partial(_seg_body, with_cnt),
                     out_type=out_type, mesh=mesh, scratch_types=scratch)


def _lin_body(x_ref, w_ref, b_ref, y_ref):
    y_ref[...] = (jnp.dot(x_ref[...], w_ref[...],
                          preferred_element_type=jnp.float32) + b_ref[...])


def _mid_body(seg_ref, cnt_ref, r1_ref, wl_ref, h_ref, y_ref):
    seg = seg_ref[0] + seg_ref[1]
    c = cnt_ref[0] + cnt_ref[1]
    inv = 1.0 / jnp.maximum(c, 1.0)
    h = jnp.maximum(seg * inv + r1_ref[...], 0.0)
    h_ref[...] = h
    y_ref[...] = jnp.dot(h, wl_ref[...], preferred_element_type=jnp.float32)


def _fin_body(seg_ref, cnt_ref, r2_ref, out_ref):
    seg = seg_ref[0] + seg_ref[1]
    c = cnt_ref[0] + cnt_ref[1]
    inv = 1.0 / jnp.maximum(c, 1.0)
    out_ref[...] = seg * inv + r2_ref[...]


_rowspec = pl.BlockSpec((_ROWBLK, D_FEAT), lambda i: (i, 0))
_wspec = pl.BlockSpec((D_FEAT, HIDDEN), lambda i: (0, 0))
_bspec = pl.BlockSpec((1, HIDDEN), lambda i: (0, 0))
_segspec = pl.BlockSpec((_NC, _ROWBLK, D_FEAT), lambda i: (0, i, 0))
_cntspec = pl.BlockSpec((_NC, _ROWBLK, 1), lambda i: (0, i, 0))
_out2 = [jax.ShapeDtypeStruct((N_NODES, HIDDEN), jnp.float32)] * 2

_lin = pl.pallas_call(
    _lin_body, grid=(_NBLK,),
    in_specs=[_rowspec, _wspec, _bspec],
    out_specs=_rowspec,
    out_shape=jax.ShapeDtypeStruct((N_NODES, HIDDEN), jnp.float32),
)

_mid = pl.pallas_call(
    _mid_body, grid=(_NBLK,),
    in_specs=[_segspec, _cntspec, _rowspec, _wspec],
    out_specs=[_rowspec, _rowspec],
    out_shape=_out2,
)

_fin = pl.pallas_call(
    _fin_body, grid=(_NBLK,),
    in_specs=[_segspec, _cntspec, _rowspec],
    out_specs=_rowspec,
    out_shape=jax.ShapeDtypeStruct((N_NODES, HIDDEN), jnp.float32),
)


def kernel(x, edge_index, W1_l, W1_r, b1, W2_l, W2_r, b2):
    src = edge_index[0].astype(jnp.int32)
    dst = edge_index[1].astype(jnp.int32)
    b1r = b1.reshape(1, HIDDEN)
    b2r = b2.reshape(1, HIDDEN)
    zb = jnp.zeros((1, HIDDEN), jnp.float32)

    y1 = _lin(x, W1_l, zb)
    seg1, cntp = _make_seg(True)(y1, src, dst)
    r1 = _lin(x, W1_r, b1r)
    cnt = cntp.reshape(_NC, _NPAD, 1)
    h, y2 = _mid(seg1, cnt, r1, W2_l)
    (seg2,) = _make_seg(False)(y2, src, dst)
    r2 = _lin(h, W2_r, b2r)
    out = _fin(seg2, cnt, r2)
    return out


# speedup vs baseline: 1.0018x; 1.0018x over previous
import functools

import jax
import jax.numpy as jnp
from jax import lax
from jax.experimental import pallas as pl
from jax.experimental.pallas import tpu as pltpu
from jax.experimental.pallas import tpu_sc as plsc

N_NODES = 10000
D_FEAT = 128
HIDDEN = 128
N_EDGES = 320000

_NC = 2
_NS = 16
_NW = _NC * _NS
_EPW = N_EDGES // _NW
_CHUNK = 80
_NCHUNK = _EPW // _CHUNK
_NPAD = 10240
_STRIPE = _NPAD // _NS

_ROWBLK = 400
_NBLK = N_NODES // _ROWBLK


def _fill_idx(idx_v, start):
    for i in range(_CHUNK // 16):
        idx_v[pl.ds(i * 16, 16)] = lax.iota(jnp.int32, 16) + start + i * 16


_DEPTH = 4


def _seg_body(with_cnt, y_hbm, src_hbm, dst_hbm, seg_out, *rest):
    if with_cnt:
        (cnt_out, acc_sh, cnt_sh, stg0, stg1, isem, didx, sidx, rows, gsem,
         ssem, ones_v, csem, i128_v, cbuf_v) = rest
    else:
        (acc_sh, stg0, stg1, isem, didx, sidx, rows, gsem, ssem) = rest
    cid = lax.axis_index("c")
    sid = lax.axis_index("s")
    wid = cid * _NS + sid
    base = sid * _STRIPE
    woff = wid * _EPW

    def _zb(i, c):
        rows[0][i // 8, pl.ds((i % 8) * 16, 16)] = jnp.zeros((16,),
                                                             jnp.float32)
        return c
    lax.fori_loop(0, _CHUNK * 8, _zb, 0)
    if with_cnt:
        for i in range(_CHUNK // 16):
            ones_v[pl.ds(i * 16, 16)] = jnp.zeros((16,), jnp.float32)
    for k in range(_STRIPE // _CHUNK):
        u = k % _DEPTH
        if k >= _DEPTH:
            pltpu.make_async_copy(rows[0], acc_sh.at[didx[u]],
                                  ssem[u]).wait()
            if with_cnt:
                pltpu.make_async_copy(ones_v, cnt_sh.at[didx[u]],
                                      csem[u]).wait()
        _fill_idx(didx[u], base + k * _CHUNK)
        pltpu.async_copy(rows[0], acc_sh.at[didx[u]], ssem[u])
        if with_cnt:
            pltpu.async_copy(ones_v, cnt_sh.at[didx[u]], csem[u])
    for u in range(_DEPTH):
        pltpu.make_async_copy(rows[0], acc_sh.at[didx[u]], ssem[u]).wait()
        if with_cnt:
            pltpu.make_async_copy(ones_v, cnt_sh.at[didx[u]],
                                  csem[u]).wait()
    if with_cnt:
        for i in range(_CHUNK // 16):
            ones_v[pl.ds(i * 16, 16)] = jnp.ones((16,), jnp.float32)

    plsc.subcore_barrier()

    blk = _DEPTH * _CHUNK
    stg = (stg0, stg1)

    def _stage(g, p):
        pltpu.async_copy(dst_hbm.at[pl.ds(woff + g * blk, blk)],
                         stg[p][0], isem[p])
        pltpu.async_copy(src_hbm.at[pl.ds(woff + g * blk, blk)],
                         stg[p][1], isem[p])

    def _stage_wait(g, p):
        pltpu.make_async_copy(dst_hbm.at[pl.ds(woff + g * blk, blk)],
                              stg[p][0], isem[p]).wait()
        pltpu.make_async_copy(src_hbm.at[pl.ds(woff + g * blk, blk)],
                              stg[p][1], isem[p]).wait()

    def _unpack(p, u):
        for i in range(_CHUNK // 16):
            didx[u][pl.ds(i * 16, 16)] = stg[p][0][pl.ds(u * _CHUNK
                                                         + i * 16, 16)]
            sidx[u][pl.ds(i * 16, 16)] = stg[p][1][pl.ds(u * _CHUNK
                                                         + i * 16, 16)]

    def _gather(u):
        pltpu.async_copy(y_hbm.at[sidx[u]], rows[u], gsem[u])

    def _gather_wait(u):
        pltpu.make_async_copy(y_hbm.at[sidx[u]], rows[u], gsem[u]).wait()

    def _scat(u):
        pltpu.async_copy(rows[u], acc_sh.at[didx[u]], ssem[u], add=True)
        if with_cnt:
            pltpu.async_copy(ones_v, cnt_sh.at[didx[u]], csem[u], add=True)

    def _scat_wait(u):
        pltpu.make_async_copy(rows[u], acc_sh.at[didx[u]], ssem[u]).wait()
        if with_cnt:
            pltpu.make_async_copy(ones_v, cnt_sh.at[didx[u]], csem[u]).wait()

    _stage(0, 0)
    _stage_wait(0, 0)
    for u in range(_DEPTH):
        _unpack(0, u)
        _gather(u)
    _stage(1, 1)
    for u in range(_DEPTH):
        _gather_wait(u)
        _scat(u)

    def _block(g, p, last=False):
        _stage_wait(g, p)
        for u in range(_DEPTH):
            _scat_wait(u)
            _unpack(p, u)
            _gather(u)
        if not last:
            _stage(g + 1, 1 - p)
        for u in range(_DEPTH):
            _gather_wait(u)
            _scat(u)

    nblk = _NCHUNK // _DEPTH

    def _body(h, c):
        g = 2 * h + 1
        _block(g, 1)
        _block(g + 1, 0)
        return c
    lax.fori_loop(0, (nblk - 3) // 2, _body, 0)
    _block(nblk - 2, 1)
    _block(nblk - 1, 0, last=True)

    for u in range(_NCHUNK % _DEPTH):
        j = nblk * _DEPTH + u
        _scat_wait(u)
        pltpu.sync_copy(dst_hbm.at[pl.ds(woff + j * _CHUNK, _CHUNK)],
                        didx[u])
        pltpu.sync_copy(src_hbm.at[pl.ds(woff + j * _CHUNK, _CHUNK)],
                        sidx[u])
        _gather(u)
    for u in range(_NCHUNK % _DEPTH, _DEPTH):
        _scat_wait(u)
    for u in range(_NCHUNK % _DEPTH):
        _gather_wait(u)
        _scat(u)
        _scat_wait(u)

    plsc.subcore_barrier()

    nw = _STRIPE // _CHUNK
    for k in range(nw):
        _fill_idx(didx[k % _DEPTH], base + k * _CHUNK)
        pltpu.async_copy(acc_sh.at[didx[k % _DEPTH]], rows[k % _DEPTH],
                         gsem[k % _DEPTH])
        if k % _DEPTH == _DEPTH - 1 or k == nw - 1:
            for kk in range(k - k % _DEPTH, k + 1):
                u = kk % _DEPTH
                pltpu.make_async_copy(acc_sh.at[didx[u]], rows[u],
                                      gsem[u]).wait()
                pltpu.sync_copy(rows[u],
                                seg_out.at[cid, pl.ds(base + kk * _CHUNK,
                                                      _CHUNK)])
    if with_cnt:
        nk = _STRIPE // 128
        for k in range(nk + 1):
            b = k % 2
            if k < nk:
                r = sid * nk + k
                for i in range(8):
                    i128_v[b][pl.ds(i * 16, 16)] = (lax.iota(jnp.int32, 16)
                                                    + r * 128 + i * 16)
                pltpu.async_copy(cnt_sh.at[i128_v[b]], cbuf_v[b], gsem[b])
            if k > 0:
                pb = 1 - b
                pltpu.make_async_copy(cnt_sh.at[i128_v[pb]], cbuf_v[pb],
                                      gsem[pb]).wait()
                pltpu.sync_copy(cbuf_v[pb],
                                cnt_out.at[cid, sid * nk + k - 1])


@functools.lru_cache(maxsize=None)
def _make_seg(with_cnt):
    mesh = plsc.VectorSubcoreMesh(core_axis_name="c", subcore_axis_name="s")
    out_type = [jax.ShapeDtypeStruct((_NC, _NPAD, D_FEAT), jnp.float32)]
    scratch = [pltpu.VMEM_SHARED((_NPAD, D_FEAT), jnp.float32)]
    if with_cnt:
        out_type.append(jax.ShapeDtypeStruct((_NC, _NPAD // 128, 128),
                                             jnp.float32))
        scratch.append(pltpu.VMEM_SHARED((_NPAD,), jnp.float32))
    scratch += [
        [pltpu.VMEM((_DEPTH * _CHUNK,), jnp.int32)] * 2,
        [pltpu.VMEM((_DEPTH * _CHUNK,), jnp.int32)] * 2,
        [pltpu.SemaphoreType.DMA] * 2,
        [pltpu.VMEM((_CHUNK,), jnp.int32)] * _DEPTH,
        [pltpu.VMEM((_CHUNK,), jnp.int32)] * _DEPTH,
        [pltpu.VMEM((_CHUNK, D_FEAT), jnp.float32)] * _DEPTH,
        [pltpu.SemaphoreType.DMA] * _DEPTH,
        [pltpu.SemaphoreType.DMA] * _DEPTH,
    ]
    if with_cnt:
        scratch += [
            pltpu.VMEM((_CHUNK,), jnp.float32),
            [pltpu.SemaphoreType.DMA] * _DEPTH,
            [pltpu.VMEM((128,), jnp.int32)] * 2,
            [pltpu.VMEM((128,), jnp.float32)] * 2,
        ]
    return pl.kernel(functools.partial(_seg_body, with_cnt),
                     out_type=out_type, mesh=mesh, scratch_types=scratch)


def _lin_body(x_ref, w_ref, b_ref, y_ref):
    y_ref[...] = (jnp.dot(x_ref[...], w_ref[...],
                          preferred_element_type=jnp.float32) + b_ref[...])


def _mid_body(seg_ref, cnt_ref, r1_ref, wl_ref, h_ref, y_ref):
    seg = seg_ref[0] + seg_ref[1]
    c = cnt_ref[0] + cnt_ref[1]
    inv = 1.0 / jnp.maximum(c, 1.0)
    h = jnp.maximum(seg * inv + r1_ref[...], 0.0)
    h_ref[...] = h
    y_ref[...] = jnp.dot(h, wl_ref[...], preferred_element_type=jnp.float32)


def _fin_body(seg_ref, cnt_ref, r2_ref, out_ref):
    seg = seg_ref[0] + seg_ref[1]
    c = cnt_ref[0] + cnt_ref[1]
    inv = 1.0 / jnp.maximum(c, 1.0)
    out_ref[...] = seg * inv + r2_ref[...]


_rowspec = pl.BlockSpec((_ROWBLK, D_FEAT), lambda i: (i, 0))
_wspec = pl.BlockSpec((D_FEAT, HIDDEN), lambda i: (0, 0))
_bspec = pl.BlockSpec((1, HIDDEN), lambda i: (0, 0))
_segspec = pl.BlockSpec((_NC, _ROWBLK, D_FEAT), lambda i: (0, i, 0))
_cntspec = pl.BlockSpec((_NC, _ROWBLK, 1), lambda i: (0, i, 0))
_out2 = [jax.ShapeDtypeStruct((N_NODES, HIDDEN), jnp.float32)] * 2

_lin = pl.pallas_call(
    _lin_body, grid=(_NBLK,),
    in_specs=[_rowspec, _wspec, _bspec],
    out_specs=_rowspec,
    out_shape=jax.ShapeDtypeStruct((N_NODES, HIDDEN), jnp.float32),
)

_mid = pl.pallas_call(
    _mid_body, grid=(_NBLK,),
    in_specs=[_segspec, _cntspec, _rowspec, _wspec],
    out_specs=[_rowspec, _rowspec],
    out_shape=_out2,
)

_fin = pl.pallas_call(
    _fin_body, grid=(_NBLK,),
    in_specs=[_segspec, _cntspec, _rowspec],
    out_specs=_rowspec,
    out_shape=jax.ShapeDtypeStruct((N_NODES, HIDDEN), jnp.float32),
)


def kernel(x, edge_index, W1_l, W1_r, b1, W2_l, W2_r, b2):
    src = edge_index[0].astype(jnp.int32)
    dst = edge_index[1].astype(jnp.int32)
    b1r = b1.reshape(1, HIDDEN)
    b2r = b2.reshape(1, HIDDEN)
    zb = jnp.zeros((1, HIDDEN), jnp.float32)

    y1 = _lin(x, W1_l, zb)
    seg1, cntp = _make_seg(True)(y1, src, dst)
    r1 = _lin(x, W1_r, b1r)
    cnt = cntp.reshape(_NC, _NPAD, 1)
    h, y2 = _mid(seg1, cnt, r1, W2_l)
    (seg2,) = _make_seg(False)(y2, src, dst)
    r2 = _lin(h, W2_r, b2r)
    out = _fin(seg2, cnt, r2)
    return out
